# baseline (speedup 1.0000x reference)
import functools

import jax
import jax.numpy as jnp
import numpy as np
from jax import lax
from jax.experimental import pallas as pl
from jax.experimental.pallas import tpu as pltpu
from jax.experimental.pallas import tpu_sc as plsc

N = 10000
E = 320000
E2 = E // 2
H = 128
HW = H // 2

NC = 2
NS = 16
NW = NC * NS
EPW = E // NW

GCH = 80
NGCH = EPW // GCH
NPAIR = NGCH // 2

SCH = 2000
NSCH = EPW // SCH

RB = 2000
BE2 = 1600

PLO = np.array([32 * (w // 16) + (w % 16) for w in range(64)], np.int32)
PHI = PLO + 16


def _tables_body(h_ref, w1a_ref, w1b_ref, b1_ref, a_ref, b_ref):
    hrows = h_ref[:]
    a_ref[:] = jnp.dot(hrows, w1a_ref[:], preferred_element_type=jnp.float32)
    b_ref[:] = jnp.dot(hrows, w1b_ref[:], preferred_element_type=jnp.float32) + b1_ref[:]


def _tables(h, w1a, w1b, b1r):
    return pl.pallas_call(
        _tables_body,
        grid=(N // RB,),
        in_specs=[
            pl.BlockSpec((RB, H), lambda i: (i, 0)),
            pl.BlockSpec((H, H), lambda i: (0, 0)),
            pl.BlockSpec((H, H), lambda i: (0, 0)),
            pl.BlockSpec((1, H), lambda i: (0, 0)),
        ],
        out_specs=[
            pl.BlockSpec((RB, H), lambda i: (i, 0)),
            pl.BlockSpec((RB, H), lambda i: (i, 0)),
        ],
        out_shape=[
            jax.ShapeDtypeStruct((N, H), jnp.float32),
            jax.ShapeDtypeStruct((N, H), jnp.float32),
        ],
    )(h, w1a, w1b, b1r)


def _addpack(buf_a, buf_b, gbuf):
    def qbody(q, carry):
        for half in range(2):
            r = 2 * q + half
            for k in range(H // 32):
                s0 = buf_a[r, pl.ds(32 * k, 16)] + buf_b[r, pl.ds(32 * k, 16)]
                s1 = buf_a[r, pl.ds(32 * k + 16, 16)] + buf_b[r, pl.ds(32 * k + 16, 16)]
                u0 = plsc.bitcast(s0, jnp.uint32)
                u1 = plsc.bitcast(s1, jnp.uint32)
                r0 = (u0 + 0x7FFF + ((u0 >> 16) & 1)) >> 16
                r1 = (u1 + 0x7FFF + ((u1 >> 16) & 1)) >> 16
                w = r0 | (r1 << 16)
                gbuf[q, pl.ds(half * 64 + 16 * k, 16)] = plsc.bitcast(w, jnp.int32)
        return carry

    lax.fori_loop(0, GCH // 2, qbody, 0)


def _gather_body(a_hbm, b_hbm, row_hbm, col_hbm, g_hbm,
                 idx_a0, idx_b0, idx_a1, idx_b1,
                 buf_a0, buf_b0, buf_a1, buf_b1,
                 gbuf0, gbuf1,
                 si_a0, si_b0, si_a1, si_b1,
                 sg_a0, sg_b0, sg_a1, sg_b1, sw0, sw1):
    wid = lax.axis_index("s") * NC + lax.axis_index("c")
    base = wid * EPW
    GR = GCH // 2

    def pair(i, carry):
        off0 = pl.multiple_of(base + (2 * i) * GCH, 8)
        off1 = pl.multiple_of(base + (2 * i + 1) * GCH, 8)

        @pl.when(i > 0)
        def _():
            pltpu.make_async_copy(gbuf0, g_hbm.at[pl.ds(0, GR)], sw0).wait()
            pltpu.make_async_copy(gbuf1, g_hbm.at[pl.ds(0, GR)], sw1).wait()

        di_a0 = pltpu.async_copy(row_hbm.at[pl.ds(off0, GCH)], idx_a0, si_a0)
        di_b0 = pltpu.async_copy(col_hbm.at[pl.ds(off0, GCH)], idx_b0, si_b0)
        di_a1 = pltpu.async_copy(row_hbm.at[pl.ds(off1, GCH)], idx_a1, si_a1)
        di_b1 = pltpu.async_copy(col_hbm.at[pl.ds(off1, GCH)], idx_b1, si_b1)
        di_a0.wait()
        di_b0.wait()
        dg_a0 = pltpu.async_copy(a_hbm.at[idx_a0], buf_a0, sg_a0)
        dg_b0 = pltpu.async_copy(b_hbm.at[idx_b0], buf_b0, sg_b0)
        di_a1.wait()
        di_b1.wait()
        dg_a1 = pltpu.async_copy(a_hbm.at[idx_a1], buf_a1, sg_a1)
        dg_b1 = pltpu.async_copy(b_hbm.at[idx_b1], buf_b1, sg_b1)
        dg_a0.wait()
        dg_b0.wait()
        _addpack(buf_a0, buf_b0, gbuf0)
        pltpu.async_copy(gbuf0, g_hbm.at[pl.ds(pl.multiple_of(off0 // 2, 8), GR)], sw0)
        dg_a1.wait()
        dg_b1.wait()
        _addpack(buf_a1, buf_b1, gbuf1)
        pltpu.async_copy(gbuf1, g_hbm.at[pl.ds(pl.multiple_of(off1 // 2, 8), GR)], sw1)
        return carry

    lax.fori_loop(0, NPAIR, pair, 0)

    pltpu.make_async_copy(gbuf0, g_hbm.at[pl.ds(0, GR)], sw0).wait()
    pltpu.make_async_copy(gbuf1, g_hbm.at[pl.ds(0, GR)], sw1).wait()

    offp = pl.multiple_of(base + (NGCH - 1) * GCH, 8)
    pltpu.sync_copy(row_hbm.at[pl.ds(offp, GCH)], idx_a0)
    pltpu.sync_copy(col_hbm.at[pl.ds(offp, GCH)], idx_b0)
    pltpu.async_copy(a_hbm.at[idx_a0], buf_a0, sg_a0).wait()
    pltpu.async_copy(b_hbm.at[idx_b0], buf_b0, sg_b0).wait()
    _addpack(buf_a0, buf_b0, gbuf0)
    pltpu.sync_copy(gbuf0, g_hbm.at[pl.ds(pl.multiple_of(offp // 2, 8), GR)])


def _gather(a_tab, b_tab, row, col):
    mesh = plsc.VectorSubcoreMesh(core_axis_name="c", subcore_axis_name="s")
    return pl.kernel(
        _gather_body,
        out_type=jax.ShapeDtypeStruct((E2, H), jnp.int32),
        mesh=mesh,
        compiler_params=pltpu.CompilerParams(needs_layout_passes=False),
        scratch_types=[
            pltpu.VMEM((GCH,), jnp.int32),
            pltpu.VMEM((GCH,), jnp.int32),
            pltpu.VMEM((GCH,), jnp.int32),
            pltpu.VMEM((GCH,), jnp.int32),
            pltpu.VMEM((GCH, H), jnp.float32),
            pltpu.VMEM((GCH, H), jnp.float32),
            pltpu.VMEM((GCH, H), jnp.float32),
            pltpu.VMEM((GCH, H), jnp.float32),
            pltpu.VMEM((GCH // 2, H), jnp.int32),
            pltpu.VMEM((GCH // 2, H), jnp.int32),
        ] + [pltpu.SemaphoreType.DMA] * 10,
    )(a_tab, b_tab, row, col)


def _mlp_body(g_ref, ea_ref, w1cl_ref, w1ch_ref, w2_ref, b2_ref, w3_ref, phi_ref):
    gi = g_ref[:]
    lo = jax.lax.bitcast_convert_type(gi << 16, jnp.float32)
    hi = jax.lax.bitcast_convert_type(gi & jnp.int32(-65536), jnp.float32)
    w2 = w2_ref[:]
    b2 = b2_ref[:]
    w3 = w3_ref[:]
    outs = []
    for c in range(2):
        l = lo[:, 64 * c:64 * (c + 1)]
        h = hi[:, 64 * c:64 * (c + 1)]
        eac = ea_ref[:, c:c + 1]
        x1 = jnp.concatenate(
            [jax.nn.silu(l + eac * w1cl_ref[:]),
             jax.nn.silu(h + eac * w1ch_ref[:])], axis=1).astype(jnp.bfloat16)
        x2 = jnp.dot(x1, w2, preferred_element_type=jnp.float32) + b2
        x2 = jax.nn.silu(x2).astype(jnp.bfloat16)
        outs.append(jnp.dot(x2, w3, preferred_element_type=jnp.float32))
    phi_ref[:] = jnp.concatenate(outs, axis=1)


def _mlp(g, ea2, w1cl, w1ch, w2cat, b2r, w3):
    return pl.pallas_call(
        _mlp_body,
        grid=(E2 // BE2,),
        in_specs=[
            pl.BlockSpec((BE2, H), lambda i: (i, 0)),
            pl.BlockSpec((BE2, 2), lambda i: (i, 0)),
            pl.BlockSpec((1, 64), lambda i: (0, 0)),
            pl.BlockSpec((1, 64), lambda i: (0, 0)),
            pl.BlockSpec((H, H), lambda i: (0, 0)),
            pl.BlockSpec((1, H), lambda i: (0, 0)),
            pl.BlockSpec((H, 1), lambda i: (0, 0)),
        ],
        out_specs=pl.BlockSpec((BE2, 2), lambda i: (i, 0)),
        out_shape=jax.ShapeDtypeStruct((E2, 2), jnp.float32),
    )(g, ea2, w1cl, w1ch, w2cat, b2r, w3)


def _scatter_body(phi_hbm, cdt_hbm, row_hbm, part_hbm,
                  acc, idx_v, phi_v, cdx_v, cdy_v, cdz_v):
    wid = lax.axis_index("s") * NC + lax.axis_index("c")
    base = wid * EPW

    zeros16 = jnp.zeros((16,), jnp.float32)

    def zbody(i, carry):
        acc[pl.ds(i * 16, 16)] = zeros16
        return carry

    lax.fori_loop(0, (3 * N) // 16, zbody, 0)

    def chunk(i, carry):
        off = pl.multiple_of(base + i * SCH, 8)
        pltpu.sync_copy(row_hbm.at[pl.ds(off, SCH)], idx_v)
        pltpu.sync_copy(phi_hbm.at[pl.ds(off, SCH)], phi_v)
        pltpu.sync_copy(cdt_hbm.at[pl.ds(off, SCH)], cdx_v)
        pltpu.sync_copy(cdt_hbm.at[pl.ds(E + off, SCH)], cdy_v)
        pltpu.sync_copy(cdt_hbm.at[pl.ds(2 * E + off, SCH)], cdz_v)

        def group(j, c2):
            sl = pl.ds(j * 16, 16)
            ii = idx_v[sl]
            p = phi_v[sl]
            plsc.addupdate_scatter(acc, [ii], cdx_v[sl] * p)
            plsc.addupdate_scatter(acc, [ii + N], cdy_v[sl] * p)
            plsc.addupdate_scatter(acc, [ii + 2 * N], cdz_v[sl] * p)
            return c2

        lax.fori_loop(0, SCH // 16, group, 0)
        return carry

    lax.fori_loop(0, NSCH, chunk, 0)
    pltpu.sync_copy(acc, part_hbm.at[wid])


def _scatter(phi_flat, cdt_flat, row):
    mesh = plsc.VectorSubcoreMesh(core_axis_name="c", subcore_axis_name="s")
    return pl.kernel(
        _scatter_body,
        out_type=jax.ShapeDtypeStruct((NW, 3 * N), jnp.float32),
        mesh=mesh,
        compiler_params=pltpu.CompilerParams(needs_layout_passes=False),
        scratch_types=[
            pltpu.VMEM((3 * N,), jnp.float32),
            pltpu.VMEM((SCH,), jnp.int32),
            pltpu.VMEM((SCH,), jnp.float32),
            pltpu.VMEM((SCH,), jnp.float32),
            pltpu.VMEM((SCH,), jnp.float32),
            pltpu.VMEM((SCH,), jnp.float32),
        ],
    )(phi_flat, cdt_flat, row)


def _combine_body(part_ref, coordt_ref, out_ref):
    agg = jnp.sum(part_ref[:], axis=0, keepdims=True)
    out_ref[:] = coordt_ref[:] + agg * (1.0 / 100.0)


def _combine(partials, coordt):
    return pl.pallas_call(
        _combine_body,
        in_specs=[
            pl.BlockSpec((NW, 3 * N), lambda: (0, 0)),
            pl.BlockSpec((1, 3 * N), lambda: (0, 0)),
        ],
        out_specs=pl.BlockSpec((1, 3 * N), lambda: (0, 0)),
        out_shape=jax.ShapeDtypeStruct((1, 3 * N), jnp.float32),
    )(partials, coordt)


def kernel(h, coord, edge_index, coord_diff, coord_cross, edge_attr, W1, b1, W2, b2, W3):
    row = edge_index[0]
    col = edge_index[1]
    w1a = W1[:H]
    w1b = W1[H:2 * H]
    w1c = W1[2 * H:2 * H + 1]
    b1r = b1.reshape(1, H)
    b2r = b2.reshape(1, H)

    a_tab, b_tab = _tables(h, w1a, w1b, b1r)
    g = _gather(a_tab, b_tab, row, col)

    w2cat = W2[np.concatenate([PLO, PHI])].astype(jnp.bfloat16)
    phi = _mlp(g, edge_attr.reshape(E2, 2),
               w1c[:, PLO], w1c[:, PHI],
               w2cat, b2r, W3.astype(jnp.bfloat16))

    cdt_flat = coord_diff.T.reshape(3 * E)
    partials = _scatter(phi.reshape(E), cdt_flat, row)

    coordt = coord.T.reshape(1, 3 * N)
    out_flat = _combine(partials, coordt)
    return out_flat.reshape(3, N).T


# speedup vs baseline: 1.2520x; 1.2520x over previous
import functools

import jax
import jax.numpy as jnp
import numpy as np
from jax import lax
from jax.experimental import pallas as pl
from jax.experimental.pallas import tpu as pltpu
from jax.experimental.pallas import tpu_sc as plsc

N = 10000
E = 320000
E2 = E // 2
H = 128
HW = H // 2

NC = 2
NS = 16
NW = NC * NS
EPW = E // NW

GCH = 80
NGCH = EPW // GCH
NPAIR = NGCH // 2

SCH = 2000
NSCH = EPW // SCH

RB = 2000
BE2 = 1600

PLO = np.array([32 * (w // 16) + (w % 16) for w in range(64)], np.int32)
PHI = PLO + 16
ILV = np.empty(H, np.int32)
ILV[0::2] = PLO
ILV[1::2] = PHI


def _tables_body(h_ref, w1a_ref, w1b_ref, b1_ref, a_ref, b_ref):
    hrows = h_ref[:]
    a_ref[:] = jnp.dot(hrows, w1a_ref[:], preferred_element_type=jnp.float32)
    b_ref[:] = jnp.dot(hrows, w1b_ref[:], preferred_element_type=jnp.float32) + b1_ref[:]


def _tables(h, w1a, w1b, b1r):
    return pl.pallas_call(
        _tables_body,
        grid=(N // RB,),
        in_specs=[
            pl.BlockSpec((RB, H), lambda i: (i, 0)),
            pl.BlockSpec((H, H), lambda i: (0, 0)),
            pl.BlockSpec((H, H), lambda i: (0, 0)),
            pl.BlockSpec((1, H), lambda i: (0, 0)),
        ],
        out_specs=[
            pl.BlockSpec((RB, H), lambda i: (i, 0)),
            pl.BlockSpec((RB, H), lambda i: (i, 0)),
        ],
        out_shape=[
            jax.ShapeDtypeStruct((N, H), jnp.float32),
            jax.ShapeDtypeStruct((N, H), jnp.float32),
        ],
    )(h, w1a, w1b, b1r)


def _addpack(buf_a, buf_b, gbuf):
    def qbody(q, carry):
        for half in range(2):
            r = 2 * q + half
            for k in range(H // 32):
                s0 = buf_a[r, pl.ds(32 * k, 16)] + buf_b[r, pl.ds(32 * k, 16)]
                s1 = buf_a[r, pl.ds(32 * k + 16, 16)] + buf_b[r, pl.ds(32 * k + 16, 16)]
                u0 = plsc.bitcast(s0, jnp.uint32)
                u1 = plsc.bitcast(s1, jnp.uint32)
                r0 = (u0 + 0x7FFF + ((u0 >> 16) & 1)) >> 16
                r1 = (u1 + 0x7FFF + ((u1 >> 16) & 1)) >> 16
                w = r0 | (r1 << 16)
                gbuf[q, pl.ds(half * 64 + 16 * k, 16)] = plsc.bitcast(w, jnp.int32)
        return carry

    lax.fori_loop(0, GCH // 2, qbody, 0)


NSLOT = 4


def _gather_body(a_hbm, b_hbm, row_hbm, col_hbm, g_hbm,
                 idx_a, idx_b, buf_a, buf_b, gbuf, si_a, si_b, sg_a, sg_b, sw):
    wid = lax.axis_index("s") * NC + lax.axis_index("c")
    base = wid * EPW
    GR = GCH // 2

    def coff(c):
        return pl.multiple_of(base + c * GCH, 8)

    def goff(c):
        return pl.multiple_of((base + c * GCH) // 2, 8)

    def issue_idx(c, s):
        return (pltpu.async_copy(row_hbm.at[pl.ds(coff(c), GCH)], idx_a[s], si_a[s]),
                pltpu.async_copy(col_hbm.at[pl.ds(coff(c), GCH)], idx_b[s], si_b[s]))

    def issue_gather(s):
        pltpu.async_copy(a_hbm.at[idx_a[s]], buf_a[s], sg_a[s])
        pltpu.async_copy(b_hbm.at[idx_b[s]], buf_b[s], sg_b[s])

    def wait(buf, hbm_slice, sem):
        pltpu.make_async_copy(buf, hbm_slice, sem).wait()

    for s in range(NSLOT):
        issue_idx(s, s)
    for s in range(NSLOT):
        wait(row_hbm.at[pl.ds(0, GCH)], idx_a[s], si_a[s])
        wait(col_hbm.at[pl.ds(0, GCH)], idx_b[s], si_b[s])
        issue_gather(s)

    NMAIN = (NGCH - 1) // NSLOT

    def block(j, carry):
        for s in range(NSLOT):
            c = 4 * j + s
            wait(a_hbm.at[pl.ds(0, GCH)], buf_a[s], sg_a[s])
            wait(b_hbm.at[pl.ds(0, GCH)], buf_b[s], sg_b[s])

            @pl.when(c + NSLOT <= NGCH - 1)
            def _():
                issue_idx(c + NSLOT, s)

            @pl.when(j > 0)
            def _():
                wait(gbuf[s], g_hbm.at[pl.ds(0, GR)], sw[s])

            _addpack(buf_a[s], buf_b[s], gbuf[s])
            pltpu.async_copy(gbuf[s], g_hbm.at[pl.ds(goff(c), GR)], sw[s])

            @pl.when(c + NSLOT <= NGCH - 1)
            def _():
                wait(row_hbm.at[pl.ds(0, GCH)], idx_a[s], si_a[s])
                wait(col_hbm.at[pl.ds(0, GCH)], idx_b[s], si_b[s])
                issue_gather(s)
        return carry

    lax.fori_loop(0, NMAIN, block, 0)

    cp = NGCH - 1
    wait(a_hbm.at[pl.ds(0, GCH)], buf_a[0], sg_a[0])
    wait(b_hbm.at[pl.ds(0, GCH)], buf_b[0], sg_b[0])
    wait(gbuf[0], g_hbm.at[pl.ds(0, GR)], sw[0])
    _addpack(buf_a[0], buf_b[0], gbuf[0])
    pltpu.sync_copy(gbuf[0], g_hbm.at[pl.ds(goff(cp), GR)])
    for s in range(1, NSLOT):
        wait(gbuf[s], g_hbm.at[pl.ds(0, GR)], sw[s])


def _gather(a_tab, b_tab, row, col):
    mesh = plsc.VectorSubcoreMesh(core_axis_name="c", subcore_axis_name="s")
    return pl.kernel(
        _gather_body,
        out_type=jax.ShapeDtypeStruct((E2, H), jnp.int32),
        mesh=mesh,
        compiler_params=pltpu.CompilerParams(needs_layout_passes=False),
        scratch_types=[
            [pltpu.VMEM((GCH,), jnp.int32)] * NSLOT,
            [pltpu.VMEM((GCH,), jnp.int32)] * NSLOT,
            [pltpu.VMEM((GCH, H), jnp.float32)] * NSLOT,
            [pltpu.VMEM((GCH, H), jnp.float32)] * NSLOT,
            [pltpu.VMEM((GCH // 2, H), jnp.int32)] * NSLOT,
            [pltpu.SemaphoreType.DMA] * NSLOT,
            [pltpu.SemaphoreType.DMA] * NSLOT,
            [pltpu.SemaphoreType.DMA] * NSLOT,
            [pltpu.SemaphoreType.DMA] * NSLOT,
            [pltpu.SemaphoreType.DMA] * NSLOT,
        ],
    )(a_tab, b_tab, row, col)


def _mlp_body(g_ref, ea_ref, w1cl_ref, w1ch_ref, w2_ref, b2_ref, w3_ref, phi_ref):
    gi = g_ref[:]
    lo = jax.lax.bitcast_convert_type(gi << 16, jnp.float32)
    hi = jax.lax.bitcast_convert_type(gi & jnp.int32(-65536), jnp.float32)
    w2 = w2_ref[:]
    b2 = b2_ref[:]
    w3 = w3_ref[:]
    outs = []
    for c in range(2):
        l = lo[:, 64 * c:64 * (c + 1)]
        h = hi[:, 64 * c:64 * (c + 1)]
        eac = ea_ref[:, c:c + 1]
        x1 = jnp.concatenate(
            [jax.nn.silu(l + eac * w1cl_ref[:]),
             jax.nn.silu(h + eac * w1ch_ref[:])], axis=1).astype(jnp.bfloat16)
        x2 = jnp.dot(x1, w2, preferred_element_type=jnp.float32) + b2
        x2 = jax.nn.silu(x2).astype(jnp.bfloat16)
        outs.append(jnp.dot(x2, w3, preferred_element_type=jnp.float32))
    phi_ref[:] = jnp.concatenate(outs, axis=1)


def _mlp(g, ea2, w1cl, w1ch, w2cat, b2r, w3):
    return pl.pallas_call(
        _mlp_body,
        grid=(E2 // BE2,),
        in_specs=[
            pl.BlockSpec((BE2, H), lambda i: (i, 0)),
            pl.BlockSpec((BE2, 2), lambda i: (i, 0)),
            pl.BlockSpec((1, 64), lambda i: (0, 0)),
            pl.BlockSpec((1, 64), lambda i: (0, 0)),
            pl.BlockSpec((H, H), lambda i: (0, 0)),
            pl.BlockSpec((1, H), lambda i: (0, 0)),
            pl.BlockSpec((H, 1), lambda i: (0, 0)),
        ],
        out_specs=pl.BlockSpec((BE2, 2), lambda i: (i, 0)),
        out_shape=jax.ShapeDtypeStruct((E2, 2), jnp.float32),
    )(g, ea2, w1cl, w1ch, w2cat, b2r, w3)


def _scatter_body(phi_hbm, cdt_hbm, row_hbm, part_hbm,
                  acc, idx_v, phi_v, cdx_v, cdy_v, cdz_v):
    wid = lax.axis_index("s") * NC + lax.axis_index("c")
    base = wid * EPW

    zeros16 = jnp.zeros((16,), jnp.float32)

    def zbody(i, carry):
        acc[pl.ds(i * 16, 16)] = zeros16
        return carry

    lax.fori_loop(0, (3 * N) // 16, zbody, 0)

    def chunk(i, carry):
        off = pl.multiple_of(base + i * SCH, 8)
        pltpu.sync_copy(row_hbm.at[pl.ds(off, SCH)], idx_v)
        pltpu.sync_copy(phi_hbm.at[pl.ds(off, SCH)], phi_v)
        pltpu.sync_copy(cdt_hbm.at[pl.ds(off, SCH)], cdx_v)
        pltpu.sync_copy(cdt_hbm.at[pl.ds(E + off, SCH)], cdy_v)
        pltpu.sync_copy(cdt_hbm.at[pl.ds(2 * E + off, SCH)], cdz_v)

        def group(j, c2):
            sl = pl.ds(j * 16, 16)
            ii = idx_v[sl]
            p = phi_v[sl]
            plsc.addupdate_scatter(acc, [ii], cdx_v[sl] * p)
            plsc.addupdate_scatter(acc, [ii + N], cdy_v[sl] * p)
            plsc.addupdate_scatter(acc, [ii + 2 * N], cdz_v[sl] * p)
            return c2

        lax.fori_loop(0, SCH // 16, group, 0)
        return carry

    lax.fori_loop(0, NSCH, chunk, 0)
    pltpu.sync_copy(acc, part_hbm.at[wid])


def _scatter(phi_flat, cdt_flat, row):
    mesh = plsc.VectorSubcoreMesh(core_axis_name="c", subcore_axis_name="s")
    return pl.kernel(
        _scatter_body,
        out_type=jax.ShapeDtypeStruct((NW, 3 * N), jnp.float32),
        mesh=mesh,
        compiler_params=pltpu.CompilerParams(needs_layout_passes=False),
        scratch_types=[
            pltpu.VMEM((3 * N,), jnp.float32),
            pltpu.VMEM((SCH,), jnp.int32),
            pltpu.VMEM((SCH,), jnp.float32),
            pltpu.VMEM((SCH,), jnp.float32),
            pltpu.VMEM((SCH,), jnp.float32),
            pltpu.VMEM((SCH,), jnp.float32),
        ],
    )(phi_flat, cdt_flat, row)


def _combine_body(part_ref, coordt_ref, out_ref):
    agg = jnp.sum(part_ref[:], axis=0, keepdims=True)
    out_ref[:] = coordt_ref[:] + agg * (1.0 / 100.0)


def _combine(partials, coordt):
    return pl.pallas_call(
        _combine_body,
        in_specs=[
            pl.BlockSpec((NW, 3 * N), lambda: (0, 0)),
            pl.BlockSpec((1, 3 * N), lambda: (0, 0)),
        ],
        out_specs=pl.BlockSpec((1, 3 * N), lambda: (0, 0)),
        out_shape=jax.ShapeDtypeStruct((1, 3 * N), jnp.float32),
    )(partials, coordt)


def kernel(h, coord, edge_index, coord_diff, coord_cross, edge_attr, W1, b1, W2, b2, W3):
    row = edge_index[0]
    col = edge_index[1]
    w1a = W1[:H]
    w1b = W1[H:2 * H]
    w1c = W1[2 * H:2 * H + 1]
    b1r = b1.reshape(1, H)
    b2r = b2.reshape(1, H)

    a_tab, b_tab = _tables(h, w1a, w1b, b1r)
    g = _gather(a_tab, b_tab, row, col)

    w2cat = W2[np.concatenate([PLO, PHI])].astype(jnp.bfloat16)
    phi = _mlp(g, edge_attr.reshape(E2, 2),
               w1c[:, PLO], w1c[:, PHI],
               w2cat, b2r, W3.astype(jnp.bfloat16))

    cdt_flat = coord_diff.T.reshape(3 * E)
    partials = _scatter(phi.reshape(E), cdt_flat, row)

    coordt = coord.T.reshape(1, 3 * N)
    out_flat = _combine(partials, coordt)
    return out_flat.reshape(3, N).T


# speedup vs baseline: 1.2596x; 1.0061x over previous
import functools

import jax
import jax.numpy as jnp
import numpy as np
from jax import lax
from jax.experimental import pallas as pl
from jax.experimental.pallas import tpu as pltpu
from jax.experimental.pallas import tpu_sc as plsc

N = 10000
E = 320000
E2 = E // 2
H = 128
HW = H // 2

NC = 2
NS = 16
NW = NC * NS
EPW = E // NW

GCH = 80
NGCH = EPW // GCH
NPAIR = NGCH // 2

SCH = 2000
NSCH = EPW // SCH

RB = 2000
BE2 = 1600

PLO = np.array([32 * (w // 16) + (w % 16) for w in range(64)], np.int32)
PHI = PLO + 16
ILV = np.empty(H, np.int32)
ILV[0::2] = PLO
ILV[1::2] = PHI


def _tables_body(h_ref, w1a_ref, w1b_ref, b1_ref, a_ref, b_ref):
    hrows = h_ref[:]
    a_ref[:] = jnp.dot(hrows, w1a_ref[:], preferred_element_type=jnp.float32)
    b_ref[:] = jnp.dot(hrows, w1b_ref[:], preferred_element_type=jnp.float32) + b1_ref[:]


def _tables(h, w1a, w1b, b1r):
    return pl.pallas_call(
        _tables_body,
        grid=(N // RB,),
        in_specs=[
            pl.BlockSpec((RB, H), lambda i: (i, 0)),
            pl.BlockSpec((H, H), lambda i: (0, 0)),
            pl.BlockSpec((H, H), lambda i: (0, 0)),
            pl.BlockSpec((1, H), lambda i: (0, 0)),
        ],
        out_specs=[
            pl.BlockSpec((RB, H), lambda i: (i, 0)),
            pl.BlockSpec((RB, H), lambda i: (i, 0)),
        ],
        out_shape=[
            jax.ShapeDtypeStruct((N, H), jnp.float32),
            jax.ShapeDtypeStruct((N, H), jnp.float32),
        ],
    )(h, w1a, w1b, b1r)


def _addpack(buf_a, buf_b, gbuf):
    def qbody(q, carry):
        for half in range(2):
            r = 2 * q + half
            for k in range(H // 32):
                s0 = buf_a[r, pl.ds(32 * k, 16)] + buf_b[r, pl.ds(32 * k, 16)]
                s1 = buf_a[r, pl.ds(32 * k + 16, 16)] + buf_b[r, pl.ds(32 * k + 16, 16)]
                u0 = plsc.bitcast(s0, jnp.uint32)
                u1 = plsc.bitcast(s1, jnp.uint32)
                r0 = (u0 + 0x7FFF + ((u0 >> 16) & 1)) >> 16
                r1 = (u1 + 0x7FFF + ((u1 >> 16) & 1)) >> 16
                w = r0 | (r1 << 16)
                gbuf[q, pl.ds(half * 64 + 16 * k, 16)] = plsc.bitcast(w, jnp.int32)
        return carry

    lax.fori_loop(0, GCH // 2, qbody, 0)


NSLOT = 4


def _gather_body(a_hbm, b_hbm, row_hbm, col_hbm, g_hbm,
                 idx_a, idx_b, buf_a, buf_b, gbuf, si_a, si_b, sg_a, sg_b, sw):
    wid = lax.axis_index("s") * NC + lax.axis_index("c")
    base = wid * EPW
    GR = GCH // 2

    def coff(c):
        return pl.multiple_of(base + c * GCH, 8)

    def goff(c):
        return pl.multiple_of((base + c * GCH) // 2, 8)

    def issue_idx(c, s):
        return (pltpu.async_copy(row_hbm.at[pl.ds(coff(c), GCH)], idx_a[s], si_a[s]),
                pltpu.async_copy(col_hbm.at[pl.ds(coff(c), GCH)], idx_b[s], si_b[s]))

    def issue_gather(s):
        pltpu.async_copy(a_hbm.at[idx_a[s]], buf_a[s], sg_a[s])
        pltpu.async_copy(b_hbm.at[idx_b[s]], buf_b[s], sg_b[s])

    def wait(buf, hbm_slice, sem):
        pltpu.make_async_copy(buf, hbm_slice, sem).wait()

    for s in range(NSLOT):
        issue_idx(s, s)
    for s in range(NSLOT):
        wait(row_hbm.at[pl.ds(0, GCH)], idx_a[s], si_a[s])
        wait(col_hbm.at[pl.ds(0, GCH)], idx_b[s], si_b[s])
        issue_gather(s)

    NMAIN = (NGCH - 1) // NSLOT

    def block(j, carry):
        for s in range(NSLOT):
            c = 4 * j + s
            wait(a_hbm.at[pl.ds(0, GCH)], buf_a[s], sg_a[s])
            wait(b_hbm.at[pl.ds(0, GCH)], buf_b[s], sg_b[s])

            @pl.when(c + NSLOT <= NGCH - 1)
            def _():
                issue_idx(c + NSLOT, s)

            @pl.when(j > 0)
            def _():
                wait(gbuf[s], g_hbm.at[pl.ds(0, GR)], sw[s])

            _addpack(buf_a[s], buf_b[s], gbuf[s])
            pltpu.async_copy(gbuf[s], g_hbm.at[pl.ds(goff(c), GR)], sw[s])

            @pl.when(c + NSLOT <= NGCH - 1)
            def _():
                wait(row_hbm.at[pl.ds(0, GCH)], idx_a[s], si_a[s])
                wait(col_hbm.at[pl.ds(0, GCH)], idx_b[s], si_b[s])
                issue_gather(s)
        return carry

    lax.fori_loop(0, NMAIN, block, 0)

    cp = NGCH - 1
    wait(a_hbm.at[pl.ds(0, GCH)], buf_a[0], sg_a[0])
    wait(b_hbm.at[pl.ds(0, GCH)], buf_b[0], sg_b[0])
    wait(gbuf[0], g_hbm.at[pl.ds(0, GR)], sw[0])
    _addpack(buf_a[0], buf_b[0], gbuf[0])
    pltpu.sync_copy(gbuf[0], g_hbm.at[pl.ds(goff(cp), GR)])
    for s in range(1, NSLOT):
        wait(gbuf[s], g_hbm.at[pl.ds(0, GR)], sw[s])


def _gather(a_tab, b_tab, row, col):
    mesh = plsc.VectorSubcoreMesh(core_axis_name="c", subcore_axis_name="s")
    return pl.kernel(
        _gather_body,
        out_type=jax.ShapeDtypeStruct((E2, H), jnp.int32),
        mesh=mesh,
        compiler_params=pltpu.CompilerParams(needs_layout_passes=False),
        scratch_types=[
            [pltpu.VMEM((GCH,), jnp.int32)] * NSLOT,
            [pltpu.VMEM((GCH,), jnp.int32)] * NSLOT,
            [pltpu.VMEM((GCH, H), jnp.float32)] * NSLOT,
            [pltpu.VMEM((GCH, H), jnp.float32)] * NSLOT,
            [pltpu.VMEM((GCH // 2, H), jnp.int32)] * NSLOT,
            [pltpu.SemaphoreType.DMA] * NSLOT,
            [pltpu.SemaphoreType.DMA] * NSLOT,
            [pltpu.SemaphoreType.DMA] * NSLOT,
            [pltpu.SemaphoreType.DMA] * NSLOT,
            [pltpu.SemaphoreType.DMA] * NSLOT,
        ],
    )(a_tab, b_tab, row, col)


def _mlp_body(g_ref, ea_ref, w1cl_ref, w1ch_ref, w2_ref, b2_ref, w3_ref, phi_ref):
    gi = g_ref[:]
    lo = jax.lax.bitcast_convert_type(gi << 16, jnp.float32)
    hi = jax.lax.bitcast_convert_type(gi & jnp.int32(-65536), jnp.float32)
    w2 = w2_ref[:]
    b2 = b2_ref[:]
    w3 = w3_ref[:]
    outs = []
    for c in range(2):
        l = lo[:, 64 * c:64 * (c + 1)]
        h = hi[:, 64 * c:64 * (c + 1)]
        eac = ea_ref[:, c:c + 1]
        x1 = jnp.concatenate(
            [jax.nn.silu(l + eac * w1cl_ref[:]),
             jax.nn.silu(h + eac * w1ch_ref[:])], axis=1).astype(jnp.bfloat16)
        x2 = jnp.dot(x1, w2, preferred_element_type=jnp.float32) + b2
        x2 = jax.nn.silu(x2).astype(jnp.bfloat16)
        outs.append(jnp.dot(x2, w3, preferred_element_type=jnp.float32))
    phi_ref[:] = jnp.concatenate(outs, axis=1)


def _mlp(g, ea2, w1cl, w1ch, w2cat, b2r, w3):
    return pl.pallas_call(
        _mlp_body,
        grid=(E2 // BE2,),
        in_specs=[
            pl.BlockSpec((BE2, H), lambda i: (i, 0)),
            pl.BlockSpec((BE2, 2), lambda i: (i, 0)),
            pl.BlockSpec((1, 64), lambda i: (0, 0)),
            pl.BlockSpec((1, 64), lambda i: (0, 0)),
            pl.BlockSpec((H, H), lambda i: (0, 0)),
            pl.BlockSpec((1, H), lambda i: (0, 0)),
            pl.BlockSpec((H, 1), lambda i: (0, 0)),
        ],
        out_specs=pl.BlockSpec((BE2, 2), lambda i: (i, 0)),
        out_shape=jax.ShapeDtypeStruct((E2, 2), jnp.float32),
    )(g, ea2, w1cl, w1ch, w2cat, b2r, w3)


def _scatter_body(phi_hbm, cd_hbm, row_hbm, part_hbm,
                  acc, idx_v, phi_v, cd_v):
    wid = lax.axis_index("s") * NC + lax.axis_index("c")
    base = wid * EPW

    zeros16 = jnp.zeros((16,), jnp.float32)

    def zbody(i, carry):
        acc[pl.ds(i * 16, 16)] = zeros16
        return carry

    lax.fori_loop(0, (3 * N) // 16, zbody, 0)

    lane3 = lax.iota(jnp.int32, 16) * 3

    def chunk(i, carry):
        off = pl.multiple_of(base + i * SCH, 8)
        pltpu.sync_copy(row_hbm.at[pl.ds(off, SCH)], idx_v)
        pltpu.sync_copy(phi_hbm.at[pl.ds(off, SCH)], phi_v)
        pltpu.sync_copy(cd_hbm.at[pl.ds(pl.multiple_of(3 * off, 8), 3 * SCH)], cd_v)

        def group(j, c2):
            sl = pl.ds(j * 16, 16)
            ii = idx_v[sl]
            p = phi_v[sl]
            flat16 = j * 48 + lane3
            for comp in range(3):
                cdc = plsc.load_gather(cd_v, [flat16 + comp])
                plsc.addupdate_scatter(acc, [ii + comp * N], cdc * p)
            return c2

        lax.fori_loop(0, SCH // 16, group, 0)
        return carry

    lax.fori_loop(0, NSCH, chunk, 0)
    pltpu.sync_copy(acc, part_hbm.at[wid])


def _scatter(phi_flat, cd, row):
    mesh = plsc.VectorSubcoreMesh(core_axis_name="c", subcore_axis_name="s")
    return pl.kernel(
        _scatter_body,
        out_type=jax.ShapeDtypeStruct((NW, 3 * N), jnp.float32),
        mesh=mesh,
        compiler_params=pltpu.CompilerParams(needs_layout_passes=False),
        scratch_types=[
            pltpu.VMEM((3 * N,), jnp.float32),
            pltpu.VMEM((SCH,), jnp.int32),
            pltpu.VMEM((SCH,), jnp.float32),
            pltpu.VMEM((3 * SCH,), jnp.float32),
        ],
    )(phi_flat, cd, row)


def _combine_body(part_ref, coordt_ref, out_ref):
    agg = jnp.sum(part_ref[:], axis=0, keepdims=True)
    out_ref[:] = coordt_ref[:] + agg * (1.0 / 100.0)


def _combine(partials, coordt):
    return pl.pallas_call(
        _combine_body,
        in_specs=[
            pl.BlockSpec((NW, 3 * N), lambda: (0, 0)),
            pl.BlockSpec((1, 3 * N), lambda: (0, 0)),
        ],
        out_specs=pl.BlockSpec((1, 3 * N), lambda: (0, 0)),
        out_shape=jax.ShapeDtypeStruct((1, 3 * N), jnp.float32),
    )(partials, coordt)


def kernel(h, coord, edge_index, coord_diff, coord_cross, edge_attr, W1, b1, W2, b2, W3):
    row = edge_index[0]
    col = edge_index[1]
    w1a = W1[:H]
    w1b = W1[H:2 * H]
    w1c = W1[2 * H:2 * H + 1]
    b1r = b1.reshape(1, H)
    b2r = b2.reshape(1, H)

    a_tab, b_tab = _tables(h, w1a, w1b, b1r)
    g = _gather(a_tab, b_tab, row, col)

    w2cat = W2[np.concatenate([PLO, PHI])].astype(jnp.bfloat16)
    phi = _mlp(g, edge_attr.reshape(E2, 2),
               w1c[:, PLO], w1c[:, PHI],
               w2cat, b2r, W3.astype(jnp.bfloat16))

    partials = _scatter(phi.reshape(E), coord_diff.reshape(3 * E), row)

    coordt = coord.T.reshape(1, 3 * N)
    out_flat = _combine(partials, coordt)
    return out_flat.reshape(3, N).T
